# baseline (speedup 1.0000x reference)
import functools

import jax
import jax.numpy as jnp
from jax import lax
from jax.experimental import pallas as pl
from jax.experimental.pallas import tpu as pltpu
from jax.experimental.pallas import tpu_sc as plsc

N = 10000
IN_DIM = 12
HID = 32
ENC = 256
SPACE = 2
K = 16
OUT_ID = 6
OUT_P4 = 6

NP = 10240
MLP_BLK = 1280


def _sh_body(x_ref, Ws_ref, bs_ref, Wh_ref, bh_ref, s_ref, h_ref):
    x = x_ref[...]
    s_ref[...] = jnp.dot(x, Ws_ref[...], preferred_element_type=jnp.float32) + bs_ref[...]
    h_ref[...] = jnp.dot(x, Wh_ref[...], preferred_element_type=jnp.float32) + bh_ref[...]


def _space_hidden(x, Ws, bs, Wh, bh):
    return pl.pallas_call(
        _sh_body,
        out_shape=(
            jax.ShapeDtypeStruct((N, SPACE), jnp.float32),
            jax.ShapeDtypeStruct((N, HID), jnp.float32),
        ),
    )(x, Ws, bs.reshape(1, SPACE), Wh, bh.reshape(1, HID))


def _mlp_body(x_ref, agg_ref, Wo1_ref, Wo2_ref, bo2_ref, W20_ref, b20_ref,
              W21_ref, b21_ref, W22_ref, b22_ref, W23_ref, b23_ref,
              W30a_ref, W30b_ref, b30_ref, W31_ref, b31_ref, W32_ref, b32_ref,
              W33_ref, b33_ref, ids_ref, p4_ref):
    lrelu = lambda v: jax.nn.leaky_relu(v, 0.01)
    dot = lambda a, b: jnp.dot(a, b, preferred_element_type=jnp.float32)
    x = x_ref[...]
    agg = agg_ref[...]
    enc = dot(x, Wo1_ref[...]) + dot(agg, Wo2_ref[...]) + bo2_ref[...]
    x1 = lrelu(enc)
    hh = lrelu(dot(x1, W20_ref[...]) + b20_ref[...])
    hh = lrelu(dot(hh, W21_ref[...]) + b21_ref[...])
    hh = lrelu(dot(hh, W22_ref[...]) + b22_ref[...])
    cand_ids = dot(hh, W23_ref[...]) + b23_ref[...]
    gg = lrelu(dot(x1, W30a_ref[...]) + dot(cand_ids, W30b_ref[...]) + b30_ref[...])
    gg = lrelu(dot(gg, W31_ref[...]) + b31_ref[...])
    gg = lrelu(dot(gg, W32_ref[...]) + b32_ref[...])
    ids_ref[...] = cand_ids
    p4_ref[...] = dot(gg, W33_ref[...]) + b33_ref[...]


def _mlp(xp, aggp, Wo1, Wo2, bo2, W20, b20, W21, b21, W22, b22, W23, b23,
         W30, b30, W31, b31, W32, b32, W33, b33):
    W30a = W30[:ENC]
    W30b = W30[ENC:]
    grid = NP // MLP_BLK
    row_spec = lambda width: pl.BlockSpec((MLP_BLK, width), lambda i: (i, 0))
    full = lambda a: pl.BlockSpec(a.shape, lambda i: (0,) * a.ndim)
    weights = [Wo1, Wo2, bo2.reshape(1, ENC), W20, b20.reshape(1, HID),
               W21, b21.reshape(1, HID), W22, b22.reshape(1, HID),
               W23, b23.reshape(1, OUT_ID), W30a, W30b, b30.reshape(1, HID),
               W31, b31.reshape(1, HID), W32, b32.reshape(1, HID),
               W33, b33.reshape(1, OUT_P4)]
    return pl.pallas_call(
        _mlp_body,
        grid=(grid,),
        in_specs=[row_spec(IN_DIM), row_spec(2 * HID)] + [full(w) for w in weights],
        out_specs=(row_spec(OUT_ID), row_spec(OUT_P4)),
        out_shape=(
            jax.ShapeDtypeStruct((NP, OUT_ID), jnp.float32),
            jax.ShapeDtypeStruct((NP, OUT_P4), jnp.float32),
        ),
    )(xp, aggp, *weights)


NW = 32
BPW = 20
QPW = 16 * BPW
NPAD = NW * QPW
NCHUNK = NPAD // 16
PAD_COORD = 1e30


QG = 8


def _knn_sc_body(sxc_hbm, syc_hbm, sxq_hbm, syq_hbm, idx_hbm, dist_hbm,
                 sxv, syv, qxv_m, qyv_m, bufi0, bufi1, bufi2, bufi3,
                 bufi4, bufi5, bufi6, bufi7, idxbuf, d2buf):
    c = lax.axis_index("c")
    s = lax.axis_index("s")
    wid = s * 2 + c
    pltpu.sync_copy(sxc_hbm, sxv)
    pltpu.sync_copy(syc_hbm, syv)
    qbase = wid * QPW
    pltpu.sync_copy(sxq_hbm.at[pl.ds(qbase, QPW)], qxv_m)
    pltpu.sync_copy(syq_hbm.at[pl.ds(qbase, QPW)], qyv_m)
    iota = lax.iota(jnp.int32, 16)
    inf = jnp.float32(jnp.inf)
    inf16 = jnp.full((16,), inf, jnp.float32)
    ones = jnp.ones((16,), jnp.int32)
    bufs = (bufi0, bufi1, bufi2, bufi3, bufi4, bufi5, bufi6, bufi7)

    def block_body(bi, _):
        boff = pl.multiple_of(bi * 16, 16)
        qxv = qxv_m[pl.ds(boff, 16)]
        qyv = qyv_m[pl.ds(boff, 16)]
        for jp in range(16 // QG):
            qs = [(qxv[QG * jp + t], qyv[QG * jp + t]) for t in range(QG)]

            def p1(ci, Rs):
                base = pl.multiple_of(ci * 16, 16)
                sxc = sxv[pl.ds(base, 16)]
                syc = syv[pl.ds(base, 16)]
                out = []
                for t in range(QG):
                    dx = qs[t][0] - sxc
                    dy = qs[t][1] - syc
                    d2 = dx * dx + dy * dy
                    out.append(jnp.minimum(Rs[t], d2))
                return tuple(out)

            Rs = lax.fori_loop(0, NCHUNK, p1, (inf16,) * QG, unroll=4)
            Ts = [plsc.cummax(R)[15] for R in Rs]

            def p2(ci, offs):
                base = pl.multiple_of(ci * 16, 16)
                sxc = sxv[pl.ds(base, 16)]
                syc = syv[pl.ds(base, 16)]
                cidx = ci * 16 + iota
                out = []
                for t in range(QG):
                    dx = qs[t][0] - sxc
                    dy = qs[t][1] - syc
                    d2 = dx * dx + dy * dy
                    m = d2 <= Ts[t]
                    incl = plsc.cumsum(ones, mask=m)
                    tgt = offs[t] + incl
                    plsc.store_scatter(bufs[t], [tgt], cidx, mask=m)
                    out.append(offs[t] + plsc.all_reduce_population_count(m))
                return tuple(out)

            minus1 = jnp.full((16,), -1, jnp.int32)
            offs = lax.fori_loop(0, NCHUNK, p2, (minus1,) * QG, unroll=4)

            for t in range(QG):
                hits = offs[t][0] + 1
                bufs[t][pl.ds(hits, 16)] = jnp.full((16,), NPAD - 1, jnp.int32)
                nb = (hits + 15) // 16

                def p3(ci, carry):
                    bk, bv = carry
                    bidx = bufs[t][pl.ds(ci * 16, 16)]
                    sxg = plsc.load_gather(sxv, [bidx])
                    syg = plsc.load_gather(syv, [bidx])
                    dx = qs[t][0] - sxg
                    dy = qs[t][1] - syg
                    d2 = dx * dx + dy * dy
                    nk, nv = plsc.sort_key_val(d2, bidx, descending=True)
                    takeold = bk <= nk
                    lk = jnp.where(takeold, bk, nk)
                    lv = jnp.where(takeold, bv, nv)
                    return tuple(plsc.sort_key_val(lk, lv))

                bk0 = inf16
                bv0 = jnp.zeros((16,), jnp.int32)
                bk, bv = lax.fori_loop(0, nb, p3, (bk0, bv0))
                ob = pl.multiple_of(boff * 16 + (QG * jp + t) * 16, 16)
                idxbuf[pl.ds(ob, 16)] = bv
                d2buf[pl.ds(ob, 16)] = bk
        return 0

    lax.fori_loop(0, BPW, block_body, 0)
    pltpu.sync_copy(idxbuf, idx_hbm.at[pl.ds(qbase * 16, QPW * 16)])
    pltpu.sync_copy(d2buf, dist_hbm.at[pl.ds(qbase * 16, QPW * 16)])


@jax.jit
def _knn_sc(sxc, syc, sxq, syq):
    mesh = plsc.VectorSubcoreMesh(core_axis_name="c", subcore_axis_name="s")
    f = pl.kernel(
        _knn_sc_body,
        out_type=(
            jax.ShapeDtypeStruct((NPAD * 16,), jnp.int32),
            jax.ShapeDtypeStruct((NPAD * 16,), jnp.float32),
        ),
        mesh=mesh,
        compiler_params=pltpu.CompilerParams(needs_layout_passes=False),
        scratch_types=[
            pltpu.VMEM((NPAD,), jnp.float32),
            pltpu.VMEM((NPAD,), jnp.float32),
            pltpu.VMEM((QPW,), jnp.float32),
            pltpu.VMEM((QPW,), jnp.float32),
            pltpu.VMEM((NPAD + 16,), jnp.int32),
            pltpu.VMEM((NPAD + 16,), jnp.int32),
            pltpu.VMEM((NPAD + 16,), jnp.int32),
            pltpu.VMEM((NPAD + 16,), jnp.int32),
            pltpu.VMEM((NPAD + 16,), jnp.int32),
            pltpu.VMEM((NPAD + 16,), jnp.int32),
            pltpu.VMEM((NPAD + 16,), jnp.int32),
            pltpu.VMEM((NPAD + 16,), jnp.int32),
            pltpu.VMEM((QPW * 16,), jnp.int32),
            pltpu.VMEM((QPW * 16,), jnp.float32),
        ],
    )
    return f(sxc, syc, sxq, syq)


def _agg_sc_body(h_hbm, idx_hbm, d2_hbm, agg_hbm, idxv, d2v, rows0, rows1,
                 aggbuf, sem0, sem1):
    c = lax.axis_index("c")
    s = lax.axis_index("s")
    wid = s * 2 + c
    qbase = wid * QPW
    pltpu.sync_copy(idx_hbm.at[pl.ds(qbase * 16, QPW * 16)], idxv)
    pltpu.sync_copy(d2_hbm.at[pl.ds(qbase * 16, QPW * 16)], d2v)

    def gather(ql, buf, sem):
        off = jnp.minimum(ql, QPW - 1) * 16
        return pltpu.async_copy(h_hbm.at[idxv.at[pl.ds(off, 16)]], buf, sem)

    def compute(ql, buf):
        d2q = d2v[pl.ds(ql * 16, 16)]
        w = jnp.exp(jnp.float32(-10.0) * d2q)
        m0 = m1 = x0 = x1 = None
        for kk in range(16):
            wk = w[kk]
            r0 = buf[kk, pl.ds(0, 16)] * wk
            r1 = buf[kk, pl.ds(16, 16)] * wk
            if kk == 0:
                m0, m1, x0, x1 = r0, r1, r0, r1
            else:
                m0 = m0 + r0
                m1 = m1 + r1
                x0 = jnp.maximum(x0, r0)
                x1 = jnp.maximum(x1, r1)
        scale = jnp.float32(1.0 / 16.0)
        ob = ql * 64
        aggbuf[pl.ds(ob, 16)] = m0 * scale
        aggbuf[pl.ds(ob + 16, 16)] = m1 * scale
        aggbuf[pl.ds(ob + 32, 16)] = x0
        aggbuf[pl.ds(ob + 48, 16)] = x1

    gather(0, rows0, sem0)
    gather(1, rows1, sem1)

    def qbody(i, _):
        q0 = i * 2
        pltpu.make_async_copy(h_hbm.at[idxv.at[pl.ds(0, 16)]], rows0, sem0).wait()
        compute(q0, rows0)
        gather(q0 + 2, rows0, sem0)
        pltpu.make_async_copy(h_hbm.at[idxv.at[pl.ds(0, 16)]], rows1, sem1).wait()
        compute(q0 + 1, rows1)
        gather(q0 + 3, rows1, sem1)
        return 0

    lax.fori_loop(0, QPW // 2, qbody, 0)
    pltpu.make_async_copy(h_hbm.at[idxv.at[pl.ds(0, 16)]], rows0, sem0).wait()
    pltpu.make_async_copy(h_hbm.at[idxv.at[pl.ds(0, 16)]], rows1, sem1).wait()
    pltpu.sync_copy(aggbuf, agg_hbm.at[pl.ds(qbase * 64, QPW * 64)])


@jax.jit
def _agg_sc(hp, idx_flat, d2_flat):
    mesh = plsc.VectorSubcoreMesh(core_axis_name="c", subcore_axis_name="s")
    f = pl.kernel(
        _agg_sc_body,
        out_type=jax.ShapeDtypeStruct((NPAD * 2 * HID,), jnp.float32),
        mesh=mesh,
        compiler_params=pltpu.CompilerParams(needs_layout_passes=False),
        scratch_types=[
            pltpu.VMEM((QPW * 16,), jnp.int32),
            pltpu.VMEM((QPW * 16,), jnp.float32),
            pltpu.VMEM((16, 128), jnp.float32),
            pltpu.VMEM((16, 128), jnp.float32),
            pltpu.VMEM((QPW * 2 * HID,), jnp.float32),
            pltpu.SemaphoreType.DMA,
            pltpu.SemaphoreType.DMA,
        ],
    )
    return f(hp, idx_flat, d2_flat)


def kernel(x, Ws, bs, Wh, bh, Wo1, Wo2, bo2, W20, b20, W21, b21, W22, b22,
           W23, b23, W30, b30, W31, b31, W32, b32, W33, b33):
    s, h = _space_hidden(x, Ws, bs, Wh, bh)
    pad = jnp.full((NPAD - N,), PAD_COORD, jnp.float32)
    zpad = jnp.zeros((NPAD - N,), jnp.float32)
    sxc = jnp.concatenate([s[:, 0], pad])
    syc = jnp.concatenate([s[:, 1], pad])
    sxq = jnp.concatenate([s[:, 0], zpad])
    syq = jnp.concatenate([s[:, 1], zpad])
    idx_flat, d2_flat = _knn_sc(sxc, syc, sxq, syq)
    idx = idx_flat.reshape(NPAD, K)[:N]
    hp = jnp.pad(h, ((0, NPAD - N), (0, 128 - HID)))
    aggp = _agg_sc(hp, idx_flat, d2_flat).reshape(NPAD, 2 * HID)
    xp = jnp.pad(x, ((0, NP - N), (0, 0)))
    ids_p, p4_p = _mlp(xp, aggp, Wo1, Wo2, bo2, W20, b20, W21, b21, W22, b22,
                       W23, b23, W30, b30, W31, b31, W32, b32, W33, b33)
    cand_ids = ids_p[:N]
    cand_p4 = p4_p[:N]
    src = idx.reshape(-1)
    dst = jnp.repeat(jnp.arange(idx.shape[0]), idx.shape[1])
    edge_index = jnp.stack([src, dst])
    return (cand_ids, cand_p4, edge_index)


# speedup vs baseline: 1.0994x; 1.0994x over previous
import functools

import jax
import jax.numpy as jnp
from jax import lax
from jax.experimental import pallas as pl
from jax.experimental.pallas import tpu as pltpu
from jax.experimental.pallas import tpu_sc as plsc

N = 10000
IN_DIM = 12
HID = 32
ENC = 256
SPACE = 2
K = 16
OUT_ID = 6
OUT_P4 = 6

NP = 10240
MLP_BLK = 1280


def _sh_body(x_ref, Ws_ref, bs_ref, Wh_ref, bh_ref, s_ref, h_ref):
    x = x_ref[...]
    s_ref[...] = jnp.dot(x, Ws_ref[...], preferred_element_type=jnp.float32) + bs_ref[...]
    h_ref[...] = jnp.dot(x, Wh_ref[...], preferred_element_type=jnp.float32) + bh_ref[...]


def _space_hidden(x, Ws, bs, Wh, bh):
    return pl.pallas_call(
        _sh_body,
        out_shape=(
            jax.ShapeDtypeStruct((N, SPACE), jnp.float32),
            jax.ShapeDtypeStruct((N, HID), jnp.float32),
        ),
    )(x, Ws, bs.reshape(1, SPACE), Wh, bh.reshape(1, HID))


def _mlp_body(x_ref, agg_ref, Wo1_ref, Wo2_ref, bo2_ref, W20_ref, b20_ref,
              W21_ref, b21_ref, W22_ref, b22_ref, W23_ref, b23_ref,
              W30a_ref, W30b_ref, b30_ref, W31_ref, b31_ref, W32_ref, b32_ref,
              W33_ref, b33_ref, ids_ref, p4_ref):
    lrelu = lambda v: jax.nn.leaky_relu(v, 0.01)
    dot = lambda a, b: jnp.dot(a, b, preferred_element_type=jnp.float32)
    x = x_ref[...]
    agg = agg_ref[...]
    enc = dot(x, Wo1_ref[...]) + dot(agg, Wo2_ref[...]) + bo2_ref[...]
    x1 = lrelu(enc)
    hh = lrelu(dot(x1, W20_ref[...]) + b20_ref[...])
    hh = lrelu(dot(hh, W21_ref[...]) + b21_ref[...])
    hh = lrelu(dot(hh, W22_ref[...]) + b22_ref[...])
    cand_ids = dot(hh, W23_ref[...]) + b23_ref[...]
    gg = lrelu(dot(x1, W30a_ref[...]) + dot(cand_ids, W30b_ref[...]) + b30_ref[...])
    gg = lrelu(dot(gg, W31_ref[...]) + b31_ref[...])
    gg = lrelu(dot(gg, W32_ref[...]) + b32_ref[...])
    ids_ref[...] = cand_ids
    p4_ref[...] = dot(gg, W33_ref[...]) + b33_ref[...]


def _mlp(xp, aggp, Wo1, Wo2, bo2, W20, b20, W21, b21, W22, b22, W23, b23,
         W30, b30, W31, b31, W32, b32, W33, b33):
    W30a = W30[:ENC]
    W30b = W30[ENC:]
    grid = NP // MLP_BLK
    row_spec = lambda width: pl.BlockSpec((MLP_BLK, width), lambda i: (i, 0))
    full = lambda a: pl.BlockSpec(a.shape, lambda i: (0,) * a.ndim)
    weights = [Wo1, Wo2, bo2.reshape(1, ENC), W20, b20.reshape(1, HID),
               W21, b21.reshape(1, HID), W22, b22.reshape(1, HID),
               W23, b23.reshape(1, OUT_ID), W30a, W30b, b30.reshape(1, HID),
               W31, b31.reshape(1, HID), W32, b32.reshape(1, HID),
               W33, b33.reshape(1, OUT_P4)]
    return pl.pallas_call(
        _mlp_body,
        grid=(grid,),
        in_specs=[row_spec(IN_DIM), row_spec(2 * HID)] + [full(w) for w in weights],
        out_specs=(row_spec(OUT_ID), row_spec(OUT_P4)),
        out_shape=(
            jax.ShapeDtypeStruct((NP, OUT_ID), jnp.float32),
            jax.ShapeDtypeStruct((NP, OUT_P4), jnp.float32),
        ),
    )(xp, aggp, *weights)


NW = 32
BPW = 20
QPW = 16 * BPW
NPAD = NW * QPW
NCHUNK = NPAD // 16
PAD_COORD = 1e30


QG = 8


def _knn_sc_body(sxc_hbm, syc_hbm, sxq_hbm, syq_hbm, idx_hbm, dist_hbm,
                 sxv, syv, qxv_m, qyv_m, bufi0, bufi1, bufi2, bufi3,
                 bufi4, bufi5, bufi6, bufi7, idxbuf, d2buf):
    c = lax.axis_index("c")
    s = lax.axis_index("s")
    wid = s * 2 + c
    pltpu.sync_copy(sxc_hbm, sxv)
    pltpu.sync_copy(syc_hbm, syv)
    qbase = wid * QPW
    pltpu.sync_copy(sxq_hbm.at[pl.ds(qbase, QPW)], qxv_m)
    pltpu.sync_copy(syq_hbm.at[pl.ds(qbase, QPW)], qyv_m)
    iota = lax.iota(jnp.int32, 16)
    inf = jnp.float32(jnp.inf)
    inf16 = jnp.full((16,), inf, jnp.float32)
    ones = jnp.ones((16,), jnp.int32)
    bufs = (bufi0, bufi1, bufi2, bufi3, bufi4, bufi5, bufi6, bufi7)

    def block_body(bi, _):
        boff = pl.multiple_of(bi * 16, 16)
        qxv = qxv_m[pl.ds(boff, 16)]
        qyv = qyv_m[pl.ds(boff, 16)]
        for jp in range(16 // QG):
            qs = [(qxv[QG * jp + t], qyv[QG * jp + t]) for t in range(QG)]

            def p1(ci, Rs):
                base = pl.multiple_of(ci * 16, 16)
                sxc = sxv[pl.ds(base, 16)]
                syc = syv[pl.ds(base, 16)]
                out = []
                for t in range(QG):
                    dx = qs[t][0] - sxc
                    dy = qs[t][1] - syc
                    d2 = dx * dx + dy * dy
                    out.append(jnp.minimum(Rs[t], d2))
                return tuple(out)

            Rs = lax.fori_loop(0, NCHUNK, p1, (inf16,) * QG, unroll=2)
            Ts = [plsc.cummax(R)[15] for R in Rs]

            def p2(ci, offs):
                base = pl.multiple_of(ci * 16, 16)
                sxc = sxv[pl.ds(base, 16)]
                syc = syv[pl.ds(base, 16)]
                cidx = ci * 16 + iota
                out = []
                for t in range(QG):
                    dx = qs[t][0] - sxc
                    dy = qs[t][1] - syc
                    d2 = dx * dx + dy * dy
                    m = d2 <= Ts[t]
                    incl = plsc.cumsum(ones, mask=m)
                    tgt = offs[t] + incl
                    plsc.store_scatter(bufs[t], [tgt], cidx, mask=m)
                    out.append(offs[t] + plsc.all_reduce_population_count(m))
                return tuple(out)

            minus1 = jnp.full((16,), -1, jnp.int32)
            offs = lax.fori_loop(0, NCHUNK, p2, (minus1,) * QG, unroll=2)

            for t in range(QG):
                hits = offs[t][0] + 1
                bufs[t][pl.ds(hits, 16)] = jnp.full((16,), NPAD - 1, jnp.int32)
                nb = (hits + 15) // 16

                def p3(ci, carry):
                    bk, bv = carry
                    bidx = bufs[t][pl.ds(ci * 16, 16)]
                    sxg = plsc.load_gather(sxv, [bidx])
                    syg = plsc.load_gather(syv, [bidx])
                    dx = qs[t][0] - sxg
                    dy = qs[t][1] - syg
                    d2 = dx * dx + dy * dy
                    nk, nv = plsc.sort_key_val(d2, bidx, descending=True)
                    takeold = bk <= nk
                    lk = jnp.where(takeold, bk, nk)
                    lv = jnp.where(takeold, bv, nv)
                    return tuple(plsc.sort_key_val(lk, lv))

                bk0 = inf16
                bv0 = jnp.zeros((16,), jnp.int32)
                bk, bv = lax.fori_loop(0, nb, p3, (bk0, bv0))
                ob = pl.multiple_of(boff * 16 + (QG * jp + t) * 16, 16)
                idxbuf[pl.ds(ob, 16)] = bv
                d2buf[pl.ds(ob, 16)] = bk
        return 0

    lax.fori_loop(0, BPW, block_body, 0)
    pltpu.sync_copy(idxbuf, idx_hbm.at[pl.ds(qbase * 16, QPW * 16)])
    pltpu.sync_copy(d2buf, dist_hbm.at[pl.ds(qbase * 16, QPW * 16)])


@jax.jit
def _knn_sc(sxc, syc, sxq, syq):
    mesh = plsc.VectorSubcoreMesh(core_axis_name="c", subcore_axis_name="s")
    f = pl.kernel(
        _knn_sc_body,
        out_type=(
            jax.ShapeDtypeStruct((NPAD * 16,), jnp.int32),
            jax.ShapeDtypeStruct((NPAD * 16,), jnp.float32),
        ),
        mesh=mesh,
        compiler_params=pltpu.CompilerParams(needs_layout_passes=False),
        scratch_types=[
            pltpu.VMEM((NPAD,), jnp.float32),
            pltpu.VMEM((NPAD,), jnp.float32),
            pltpu.VMEM((QPW,), jnp.float32),
            pltpu.VMEM((QPW,), jnp.float32),
            pltpu.VMEM((NPAD + 16,), jnp.int32),
            pltpu.VMEM((NPAD + 16,), jnp.int32),
            pltpu.VMEM((NPAD + 16,), jnp.int32),
            pltpu.VMEM((NPAD + 16,), jnp.int32),
            pltpu.VMEM((NPAD + 16,), jnp.int32),
            pltpu.VMEM((NPAD + 16,), jnp.int32),
            pltpu.VMEM((NPAD + 16,), jnp.int32),
            pltpu.VMEM((NPAD + 16,), jnp.int32),
            pltpu.VMEM((QPW * 16,), jnp.int32),
            pltpu.VMEM((QPW * 16,), jnp.float32),
        ],
    )
    return f(sxc, syc, sxq, syq)


def _agg_sc_body(h_hbm, idx_hbm, d2_hbm, agg_hbm, idxv, d2v, rows0, rows1,
                 aggbuf, sem0, sem1):
    c = lax.axis_index("c")
    s = lax.axis_index("s")
    wid = s * 2 + c
    qbase = wid * QPW
    pltpu.sync_copy(idx_hbm.at[pl.ds(qbase * 16, QPW * 16)], idxv)
    pltpu.sync_copy(d2_hbm.at[pl.ds(qbase * 16, QPW * 16)], d2v)

    def gather(ql, buf, sem):
        off = jnp.minimum(ql, QPW - 1) * 16
        return pltpu.async_copy(h_hbm.at[idxv.at[pl.ds(off, 16)]], buf, sem)

    def compute(ql, buf):
        d2q = d2v[pl.ds(ql * 16, 16)]
        w = jnp.exp(jnp.float32(-10.0) * d2q)
        m0 = m1 = x0 = x1 = None
        for kk in range(16):
            wk = w[kk]
            r0 = buf[kk, pl.ds(0, 16)] * wk
            r1 = buf[kk, pl.ds(16, 16)] * wk
            if kk == 0:
                m0, m1, x0, x1 = r0, r1, r0, r1
            else:
                m0 = m0 + r0
                m1 = m1 + r1
                x0 = jnp.maximum(x0, r0)
                x1 = jnp.maximum(x1, r1)
        scale = jnp.float32(1.0 / 16.0)
        ob = ql * 64
        aggbuf[pl.ds(ob, 16)] = m0 * scale
        aggbuf[pl.ds(ob + 16, 16)] = m1 * scale
        aggbuf[pl.ds(ob + 32, 16)] = x0
        aggbuf[pl.ds(ob + 48, 16)] = x1

    gather(0, rows0, sem0)
    gather(1, rows1, sem1)

    def qbody(i, _):
        q0 = i * 2
        pltpu.make_async_copy(h_hbm.at[idxv.at[pl.ds(0, 16)]], rows0, sem0).wait()
        compute(q0, rows0)
        gather(q0 + 2, rows0, sem0)
        pltpu.make_async_copy(h_hbm.at[idxv.at[pl.ds(0, 16)]], rows1, sem1).wait()
        compute(q0 + 1, rows1)
        gather(q0 + 3, rows1, sem1)
        return 0

    lax.fori_loop(0, QPW // 2, qbody, 0)
    pltpu.make_async_copy(h_hbm.at[idxv.at[pl.ds(0, 16)]], rows0, sem0).wait()
    pltpu.make_async_copy(h_hbm.at[idxv.at[pl.ds(0, 16)]], rows1, sem1).wait()
    pltpu.sync_copy(aggbuf, agg_hbm.at[pl.ds(qbase * 64, QPW * 64)])


@jax.jit
def _agg_sc(hp, idx_flat, d2_flat):
    mesh = plsc.VectorSubcoreMesh(core_axis_name="c", subcore_axis_name="s")
    f = pl.kernel(
        _agg_sc_body,
        out_type=jax.ShapeDtypeStruct((NPAD * 2 * HID,), jnp.float32),
        mesh=mesh,
        compiler_params=pltpu.CompilerParams(needs_layout_passes=False),
        scratch_types=[
            pltpu.VMEM((QPW * 16,), jnp.int32),
            pltpu.VMEM((QPW * 16,), jnp.float32),
            pltpu.VMEM((16, 128), jnp.float32),
            pltpu.VMEM((16, 128), jnp.float32),
            pltpu.VMEM((QPW * 2 * HID,), jnp.float32),
            pltpu.SemaphoreType.DMA,
            pltpu.SemaphoreType.DMA,
        ],
    )
    return f(hp, idx_flat, d2_flat)


def kernel(x, Ws, bs, Wh, bh, Wo1, Wo2, bo2, W20, b20, W21, b21, W22, b22,
           W23, b23, W30, b30, W31, b31, W32, b32, W33, b33):
    s, h = _space_hidden(x, Ws, bs, Wh, bh)
    pad = jnp.full((NPAD - N,), PAD_COORD, jnp.float32)
    zpad = jnp.zeros((NPAD - N,), jnp.float32)
    sxc = jnp.concatenate([s[:, 0], pad])
    syc = jnp.concatenate([s[:, 1], pad])
    sxq = jnp.concatenate([s[:, 0], zpad])
    syq = jnp.concatenate([s[:, 1], zpad])
    idx_flat, d2_flat = _knn_sc(sxc, syc, sxq, syq)
    idx = idx_flat.reshape(NPAD, K)[:N]
    hp = jnp.pad(h, ((0, NPAD - N), (0, 128 - HID)))
    aggp = _agg_sc(hp, idx_flat, d2_flat).reshape(NPAD, 2 * HID)
    xp = jnp.pad(x, ((0, NP - N), (0, 0)))
    ids_p, p4_p = _mlp(xp, aggp, Wo1, Wo2, bo2, W20, b20, W21, b21, W22, b22,
                       W23, b23, W30, b30, W31, b31, W32, b32, W33, b33)
    cand_ids = ids_p[:N]
    cand_p4 = p4_p[:N]
    src = idx.reshape(-1)
    dst = jnp.repeat(jnp.arange(idx.shape[0]), idx.shape[1])
    edge_index = jnp.stack([src, dst])
    return (cand_ids, cand_p4, edge_index)


# speedup vs baseline: 1.2726x; 1.1576x over previous
import functools

import jax
import jax.numpy as jnp
from jax import lax
from jax.experimental import pallas as pl
from jax.experimental.pallas import tpu as pltpu
from jax.experimental.pallas import tpu_sc as plsc

N = 10000
IN_DIM = 12
HID = 32
ENC = 256
SPACE = 2
K = 16
OUT_ID = 6
OUT_P4 = 6

NP = 10240
MLP_BLK = 1280


def _sh_body(x_ref, Ws_ref, bs_ref, Wh_ref, bh_ref, s_ref, h_ref):
    x = x_ref[...]
    s_ref[...] = jnp.dot(x, Ws_ref[...], preferred_element_type=jnp.float32) + bs_ref[...]
    h_ref[...] = jnp.dot(x, Wh_ref[...], preferred_element_type=jnp.float32) + bh_ref[...]


def _space_hidden(x, Ws, bs, Wh, bh):
    return pl.pallas_call(
        _sh_body,
        out_shape=(
            jax.ShapeDtypeStruct((N, SPACE), jnp.float32),
            jax.ShapeDtypeStruct((N, HID), jnp.float32),
        ),
    )(x, Ws, bs.reshape(1, SPACE), Wh, bh.reshape(1, HID))


def _mlp_body(x_ref, agg_ref, Wo1_ref, Wo2_ref, bo2_ref, W20_ref, b20_ref,
              W21_ref, b21_ref, W22_ref, b22_ref, W23_ref, b23_ref,
              W30a_ref, W30b_ref, b30_ref, W31_ref, b31_ref, W32_ref, b32_ref,
              W33_ref, b33_ref, ids_ref, p4_ref):
    lrelu = lambda v: jax.nn.leaky_relu(v, 0.01)
    dot = lambda a, b: jnp.dot(a, b, preferred_element_type=jnp.float32)
    x = x_ref[...]
    agg = agg_ref[...]
    enc = dot(x, Wo1_ref[...]) + dot(agg, Wo2_ref[...]) + bo2_ref[...]
    x1 = lrelu(enc)
    hh = lrelu(dot(x1, W20_ref[...]) + b20_ref[...])
    hh = lrelu(dot(hh, W21_ref[...]) + b21_ref[...])
    hh = lrelu(dot(hh, W22_ref[...]) + b22_ref[...])
    cand_ids = dot(hh, W23_ref[...]) + b23_ref[...]
    gg = lrelu(dot(x1, W30a_ref[...]) + dot(cand_ids, W30b_ref[...]) + b30_ref[...])
    gg = lrelu(dot(gg, W31_ref[...]) + b31_ref[...])
    gg = lrelu(dot(gg, W32_ref[...]) + b32_ref[...])
    ids_ref[...] = cand_ids
    p4_ref[...] = dot(gg, W33_ref[...]) + b33_ref[...]


def _mlp(xp, aggp, Wo1, Wo2, bo2, W20, b20, W21, b21, W22, b22, W23, b23,
         W30, b30, W31, b31, W32, b32, W33, b33):
    W30a = W30[:ENC]
    W30b = W30[ENC:]
    grid = NP // MLP_BLK
    row_spec = lambda width: pl.BlockSpec((MLP_BLK, width), lambda i: (i, 0))
    full = lambda a: pl.BlockSpec(a.shape, lambda i: (0,) * a.ndim)
    weights = [Wo1, Wo2, bo2.reshape(1, ENC), W20, b20.reshape(1, HID),
               W21, b21.reshape(1, HID), W22, b22.reshape(1, HID),
               W23, b23.reshape(1, OUT_ID), W30a, W30b, b30.reshape(1, HID),
               W31, b31.reshape(1, HID), W32, b32.reshape(1, HID),
               W33, b33.reshape(1, OUT_P4)]
    return pl.pallas_call(
        _mlp_body,
        grid=(grid,),
        in_specs=[row_spec(IN_DIM), row_spec(2 * HID)] + [full(w) for w in weights],
        out_specs=(row_spec(OUT_ID), row_spec(OUT_P4)),
        out_shape=(
            jax.ShapeDtypeStruct((NP, OUT_ID), jnp.float32),
            jax.ShapeDtypeStruct((NP, OUT_P4), jnp.float32),
        ),
    )(xp, aggp, *weights)


NW = 32
BPW = 20
QPW = 16 * BPW
NPAD = NW * QPW
NCHUNK = NPAD // 16
PAD_COORD = 1e30


TBLK = 256
NGRP = 16
GRPW = NPAD // NGRP


def _thresh_body(qx_ref, qy_ref, sx_ref, sy_ref, t_ref):
    qx = qx_ref[...]
    qy = qy_ref[...]
    mins = []
    for g in range(NGRP):
        sx = sx_ref[:, g * GRPW:(g + 1) * GRPW]
        sy = sy_ref[:, g * GRPW:(g + 1) * GRPW]
        dx = qx - sx
        dy = qy - sy
        d2 = dx * dx + dy * dy
        mins.append(jnp.min(d2, axis=1, keepdims=True))
    t = mins[0]
    for m in mins[1:]:
        t = jnp.maximum(t, m)
    t_ref[...] = t * jnp.float32(1.0 + 1e-5)


def _thresh_tc(sxq, syq, sxc, syc):
    qspec = pl.BlockSpec((TBLK, 1), lambda i: (i, 0))
    cspec = pl.BlockSpec((1, NPAD), lambda i: (0, 0))
    return pl.pallas_call(
        _thresh_body,
        grid=(NPAD // TBLK,),
        in_specs=[qspec, qspec, cspec, cspec],
        out_specs=qspec,
        out_shape=jax.ShapeDtypeStruct((NPAD, 1), jnp.float32),
    )(sxq.reshape(NPAD, 1), syq.reshape(NPAD, 1),
      sxc.reshape(1, NPAD), syc.reshape(1, NPAD))


QG = 8


def _knn_sc_body(sxc_hbm, syc_hbm, sxq_hbm, syq_hbm, tq_hbm, idx_hbm, dist_hbm,
                 sxv, syv, qxv_m, qyv_m, tv_m, bufi0, bufi1, bufi2, bufi3,
                 bufi4, bufi5, bufi6, bufi7, idxbuf, d2buf):
    c = lax.axis_index("c")
    s = lax.axis_index("s")
    wid = s * 2 + c
    pltpu.sync_copy(sxc_hbm, sxv)
    pltpu.sync_copy(syc_hbm, syv)
    qbase = wid * QPW
    pltpu.sync_copy(sxq_hbm.at[pl.ds(qbase, QPW)], qxv_m)
    pltpu.sync_copy(syq_hbm.at[pl.ds(qbase, QPW)], qyv_m)
    pltpu.sync_copy(tq_hbm.at[pl.ds(qbase, QPW)], tv_m)
    iota = lax.iota(jnp.int32, 16)
    inf = jnp.float32(jnp.inf)
    inf16 = jnp.full((16,), inf, jnp.float32)
    ones = jnp.ones((16,), jnp.int32)
    bufs = (bufi0, bufi1, bufi2, bufi3, bufi4, bufi5, bufi6, bufi7)

    def block_body(bi, _):
        boff = pl.multiple_of(bi * 16, 16)
        qxv = qxv_m[pl.ds(boff, 16)]
        qyv = qyv_m[pl.ds(boff, 16)]
        tqv = tv_m[pl.ds(boff, 16)]
        for jp in range(16 // QG):
            qs = [(qxv[QG * jp + t], qyv[QG * jp + t]) for t in range(QG)]
            Ts = [tqv[QG * jp + t] for t in range(QG)]

            def p2(ci, offs):
                base = pl.multiple_of(ci * 16, 16)
                sxc = sxv[pl.ds(base, 16)]
                syc = syv[pl.ds(base, 16)]
                cidx = ci * 16 + iota
                out = []
                for t in range(QG):
                    dx = qs[t][0] - sxc
                    dy = qs[t][1] - syc
                    d2 = dx * dx + dy * dy
                    m = d2 <= Ts[t]
                    incl = plsc.cumsum(ones, mask=m)
                    tgt = offs[t] + incl
                    plsc.store_scatter(bufs[t], [tgt], cidx, mask=m)
                    out.append(offs[t] + plsc.all_reduce_population_count(m))
                return tuple(out)

            minus1 = jnp.full((16,), -1, jnp.int32)
            offs = lax.fori_loop(0, NCHUNK, p2, (minus1,) * QG, unroll=2)

            for t in range(QG):
                hits = offs[t][0] + 1
                bufs[t][pl.ds(hits, 16)] = jnp.full((16,), NPAD - 1, jnp.int32)
                nb = (hits + 15) // 16

                def p3(ci, carry):
                    bk, bv = carry
                    bidx = bufs[t][pl.ds(ci * 16, 16)]
                    sxg = plsc.load_gather(sxv, [bidx])
                    syg = plsc.load_gather(syv, [bidx])
                    dx = qs[t][0] - sxg
                    dy = qs[t][1] - syg
                    d2 = dx * dx + dy * dy
                    nk, nv = plsc.sort_key_val(d2, bidx, descending=True)
                    takeold = bk <= nk
                    lk = jnp.where(takeold, bk, nk)
                    lv = jnp.where(takeold, bv, nv)
                    return tuple(plsc.sort_key_val(lk, lv))

                bk0 = inf16
                bv0 = jnp.zeros((16,), jnp.int32)
                bk, bv = lax.fori_loop(0, nb, p3, (bk0, bv0))
                ob = pl.multiple_of(boff * 16 + (QG * jp + t) * 16, 16)
                idxbuf[pl.ds(ob, 16)] = bv
                d2buf[pl.ds(ob, 16)] = bk
        return 0

    lax.fori_loop(0, BPW, block_body, 0)
    pltpu.sync_copy(idxbuf, idx_hbm.at[pl.ds(qbase * 16, QPW * 16)])
    pltpu.sync_copy(d2buf, dist_hbm.at[pl.ds(qbase * 16, QPW * 16)])


@jax.jit
def _knn_sc(sxc, syc, sxq, syq, tq):
    mesh = plsc.VectorSubcoreMesh(core_axis_name="c", subcore_axis_name="s")
    f = pl.kernel(
        _knn_sc_body,
        out_type=(
            jax.ShapeDtypeStruct((NPAD * 16,), jnp.int32),
            jax.ShapeDtypeStruct((NPAD * 16,), jnp.float32),
        ),
        mesh=mesh,
        compiler_params=pltpu.CompilerParams(needs_layout_passes=False),
        scratch_types=[
            pltpu.VMEM((NPAD,), jnp.float32),
            pltpu.VMEM((NPAD,), jnp.float32),
            pltpu.VMEM((QPW,), jnp.float32),
            pltpu.VMEM((QPW,), jnp.float32),
            pltpu.VMEM((QPW,), jnp.float32),
            pltpu.VMEM((NPAD + 16,), jnp.int32),
            pltpu.VMEM((NPAD + 16,), jnp.int32),
            pltpu.VMEM((NPAD + 16,), jnp.int32),
            pltpu.VMEM((NPAD + 16,), jnp.int32),
            pltpu.VMEM((NPAD + 16,), jnp.int32),
            pltpu.VMEM((NPAD + 16,), jnp.int32),
            pltpu.VMEM((NPAD + 16,), jnp.int32),
            pltpu.VMEM((NPAD + 16,), jnp.int32),
            pltpu.VMEM((QPW * 16,), jnp.int32),
            pltpu.VMEM((QPW * 16,), jnp.float32),
        ],
    )
    return f(sxc, syc, sxq, syq, tq)


def _agg_sc_body(h_hbm, idx_hbm, d2_hbm, agg_hbm, idxv, d2v, rows0, rows1,
                 aggbuf, sem0, sem1):
    c = lax.axis_index("c")
    s = lax.axis_index("s")
    wid = s * 2 + c
    qbase = wid * QPW
    pltpu.sync_copy(idx_hbm.at[pl.ds(qbase * 16, QPW * 16)], idxv)
    pltpu.sync_copy(d2_hbm.at[pl.ds(qbase * 16, QPW * 16)], d2v)

    def gather(ql, buf, sem):
        off = jnp.minimum(ql, QPW - 1) * 16
        return pltpu.async_copy(h_hbm.at[idxv.at[pl.ds(off, 16)]], buf, sem)

    def compute(ql, buf):
        d2q = d2v[pl.ds(ql * 16, 16)]
        w = jnp.exp(jnp.float32(-10.0) * d2q)
        m0 = m1 = x0 = x1 = None
        for kk in range(16):
            wk = w[kk]
            r0 = buf[kk, pl.ds(0, 16)] * wk
            r1 = buf[kk, pl.ds(16, 16)] * wk
            if kk == 0:
                m0, m1, x0, x1 = r0, r1, r0, r1
            else:
                m0 = m0 + r0
                m1 = m1 + r1
                x0 = jnp.maximum(x0, r0)
                x1 = jnp.maximum(x1, r1)
        scale = jnp.float32(1.0 / 16.0)
        ob = ql * 64
        aggbuf[pl.ds(ob, 16)] = m0 * scale
        aggbuf[pl.ds(ob + 16, 16)] = m1 * scale
        aggbuf[pl.ds(ob + 32, 16)] = x0
        aggbuf[pl.ds(ob + 48, 16)] = x1

    gather(0, rows0, sem0)
    gather(1, rows1, sem1)

    def qbody(i, _):
        q0 = i * 2
        pltpu.make_async_copy(h_hbm.at[idxv.at[pl.ds(0, 16)]], rows0, sem0).wait()
        compute(q0, rows0)
        gather(q0 + 2, rows0, sem0)
        pltpu.make_async_copy(h_hbm.at[idxv.at[pl.ds(0, 16)]], rows1, sem1).wait()
        compute(q0 + 1, rows1)
        gather(q0 + 3, rows1, sem1)
        return 0

    lax.fori_loop(0, QPW // 2, qbody, 0)
    pltpu.make_async_copy(h_hbm.at[idxv.at[pl.ds(0, 16)]], rows0, sem0).wait()
    pltpu.make_async_copy(h_hbm.at[idxv.at[pl.ds(0, 16)]], rows1, sem1).wait()
    pltpu.sync_copy(aggbuf, agg_hbm.at[pl.ds(qbase * 64, QPW * 64)])


@jax.jit
def _agg_sc(hp, idx_flat, d2_flat):
    mesh = plsc.VectorSubcoreMesh(core_axis_name="c", subcore_axis_name="s")
    f = pl.kernel(
        _agg_sc_body,
        out_type=jax.ShapeDtypeStruct((NPAD * 2 * HID,), jnp.float32),
        mesh=mesh,
        compiler_params=pltpu.CompilerParams(needs_layout_passes=False),
        scratch_types=[
            pltpu.VMEM((QPW * 16,), jnp.int32),
            pltpu.VMEM((QPW * 16,), jnp.float32),
            pltpu.VMEM((16, 128), jnp.float32),
            pltpu.VMEM((16, 128), jnp.float32),
            pltpu.VMEM((QPW * 2 * HID,), jnp.float32),
            pltpu.SemaphoreType.DMA,
            pltpu.SemaphoreType.DMA,
        ],
    )
    return f(hp, idx_flat, d2_flat)


def kernel(x, Ws, bs, Wh, bh, Wo1, Wo2, bo2, W20, b20, W21, b21, W22, b22,
           W23, b23, W30, b30, W31, b31, W32, b32, W33, b33):
    s, h = _space_hidden(x, Ws, bs, Wh, bh)
    pad = jnp.full((NPAD - N,), PAD_COORD, jnp.float32)
    zpad = jnp.zeros((NPAD - N,), jnp.float32)
    sxc = jnp.concatenate([s[:, 0], pad])
    syc = jnp.concatenate([s[:, 1], pad])
    sxq = jnp.concatenate([s[:, 0], zpad])
    syq = jnp.concatenate([s[:, 1], zpad])
    tq = _thresh_tc(sxq, syq, sxc, syc).reshape(NPAD)
    idx_flat, d2_flat = _knn_sc(sxc, syc, sxq, syq, tq)
    idx = idx_flat.reshape(NPAD, K)[:N]
    hp = jnp.pad(h, ((0, NPAD - N), (0, 128 - HID)))
    aggp = _agg_sc(hp, idx_flat, d2_flat).reshape(NPAD, 2 * HID)
    xp = jnp.pad(x, ((0, NP - N), (0, 0)))
    ids_p, p4_p = _mlp(xp, aggp, Wo1, Wo2, bo2, W20, b20, W21, b21, W22, b22,
                       W23, b23, W30, b30, W31, b31, W32, b32, W33, b33)
    cand_ids = ids_p[:N]
    cand_p4 = p4_p[:N]
    src = idx.reshape(-1)
    dst = jnp.repeat(jnp.arange(idx.shape[0]), idx.shape[1])
    edge_index = jnp.stack([src, dst])
    return (cand_ids, cand_p4, edge_index)


# speedup vs baseline: 1.3530x; 1.0632x over previous
import functools

import jax
import jax.numpy as jnp
from jax import lax
from jax.experimental import pallas as pl
from jax.experimental.pallas import tpu as pltpu
from jax.experimental.pallas import tpu_sc as plsc

N = 10000
IN_DIM = 12
HID = 32
ENC = 256
SPACE = 2
K = 16
OUT_ID = 6
OUT_P4 = 6

NP = 10240
MLP_BLK = 1280


def _sh_body(x_ref, Ws_ref, bs_ref, Wh_ref, bh_ref, s_ref, h_ref):
    x = x_ref[...]
    s_ref[...] = jnp.dot(x, Ws_ref[...], preferred_element_type=jnp.float32) + bs_ref[...]
    h_ref[...] = jnp.dot(x, Wh_ref[...], preferred_element_type=jnp.float32) + bh_ref[...]


def _space_hidden(x, Ws, bs, Wh, bh):
    return pl.pallas_call(
        _sh_body,
        out_shape=(
            jax.ShapeDtypeStruct((N, SPACE), jnp.float32),
            jax.ShapeDtypeStruct((N, HID), jnp.float32),
        ),
    )(x, Ws, bs.reshape(1, SPACE), Wh, bh.reshape(1, HID))


def _mlp_body(x_ref, agg_ref, Wo1_ref, Wo2_ref, bo2_ref, W20_ref, b20_ref,
              W21_ref, b21_ref, W22_ref, b22_ref, W23_ref, b23_ref,
              W30a_ref, W30b_ref, b30_ref, W31_ref, b31_ref, W32_ref, b32_ref,
              W33_ref, b33_ref, ids_ref, p4_ref):
    lrelu = lambda v: jax.nn.leaky_relu(v, 0.01)
    dot = lambda a, b: jnp.dot(a, b, preferred_element_type=jnp.float32)
    x = x_ref[...]
    agg = agg_ref[...]
    enc = dot(x, Wo1_ref[...]) + dot(agg, Wo2_ref[...]) + bo2_ref[...]
    x1 = lrelu(enc)
    hh = lrelu(dot(x1, W20_ref[...]) + b20_ref[...])
    hh = lrelu(dot(hh, W21_ref[...]) + b21_ref[...])
    hh = lrelu(dot(hh, W22_ref[...]) + b22_ref[...])
    cand_ids = dot(hh, W23_ref[...]) + b23_ref[...]
    gg = lrelu(dot(x1, W30a_ref[...]) + dot(cand_ids, W30b_ref[...]) + b30_ref[...])
    gg = lrelu(dot(gg, W31_ref[...]) + b31_ref[...])
    gg = lrelu(dot(gg, W32_ref[...]) + b32_ref[...])
    ids_ref[...] = cand_ids
    p4_ref[...] = dot(gg, W33_ref[...]) + b33_ref[...]


def _mlp(xp, aggp, Wo1, Wo2, bo2, W20, b20, W21, b21, W22, b22, W23, b23,
         W30, b30, W31, b31, W32, b32, W33, b33):
    W30a = W30[:ENC]
    W30b = W30[ENC:]
    grid = NP // MLP_BLK
    row_spec = lambda width: pl.BlockSpec((MLP_BLK, width), lambda i: (i, 0))
    full = lambda a: pl.BlockSpec(a.shape, lambda i: (0,) * a.ndim)
    weights = [Wo1, Wo2, bo2.reshape(1, ENC), W20, b20.reshape(1, HID),
               W21, b21.reshape(1, HID), W22, b22.reshape(1, HID),
               W23, b23.reshape(1, OUT_ID), W30a, W30b, b30.reshape(1, HID),
               W31, b31.reshape(1, HID), W32, b32.reshape(1, HID),
               W33, b33.reshape(1, OUT_P4)]
    return pl.pallas_call(
        _mlp_body,
        grid=(grid,),
        in_specs=[row_spec(IN_DIM), row_spec(2 * HID)] + [full(w) for w in weights],
        out_specs=(row_spec(OUT_ID), row_spec(OUT_P4)),
        out_shape=(
            jax.ShapeDtypeStruct((NP, OUT_ID), jnp.float32),
            jax.ShapeDtypeStruct((NP, OUT_P4), jnp.float32),
        ),
    )(xp, aggp, *weights)


NW = 32
BPW = 20
QPW = 16 * BPW
NPAD = NW * QPW
NCHUNK = NPAD // 16
PAD_COORD = 1e30


TBLK = 256
NGRP = 16
GRPW = NPAD // NGRP


def _thresh_body(qx_ref, qy_ref, sx_ref, sy_ref, t_ref):
    qx = qx_ref[...]
    qy = qy_ref[...]
    mins = []
    for g in range(NGRP):
        sx = sx_ref[:, g * GRPW:(g + 1) * GRPW]
        sy = sy_ref[:, g * GRPW:(g + 1) * GRPW]
        dx = qx - sx
        dy = qy - sy
        d2 = dx * dx + dy * dy
        mins.append(jnp.min(d2, axis=1, keepdims=True))
    t = mins[0]
    for m in mins[1:]:
        t = jnp.maximum(t, m)
    t_ref[...] = t * jnp.float32(1.0 + 1e-5)


def _thresh_tc(sxq, syq, sxc, syc):
    qspec = pl.BlockSpec((TBLK, 1), lambda i: (i, 0))
    cspec = pl.BlockSpec((1, NPAD), lambda i: (0, 0))
    return pl.pallas_call(
        _thresh_body,
        grid=(NPAD // TBLK,),
        in_specs=[qspec, qspec, cspec, cspec],
        out_specs=qspec,
        out_shape=jax.ShapeDtypeStruct((NPAD, 1), jnp.float32),
    )(sxq.reshape(NPAD, 1), syq.reshape(NPAD, 1),
      sxc.reshape(1, NPAD), syc.reshape(1, NPAD))


QG = 8


def _knn_sc_body(sxc_hbm, syc_hbm, sxq_hbm, syq_hbm, tq_hbm, idx_hbm, dist_hbm,
                 sxv, syv, qxv_m, qyv_m, tv_m, bufi0, bufi1, bufi2, bufi3,
                 bufi4, bufi5, bufi6, bufi7, idxbuf, d2buf):
    c = lax.axis_index("c")
    s = lax.axis_index("s")
    wid = s * 2 + c
    pltpu.sync_copy(sxc_hbm, sxv)
    pltpu.sync_copy(syc_hbm, syv)
    qbase = wid * QPW
    pltpu.sync_copy(sxq_hbm.at[pl.ds(qbase, QPW)], qxv_m)
    pltpu.sync_copy(syq_hbm.at[pl.ds(qbase, QPW)], qyv_m)
    pltpu.sync_copy(tq_hbm.at[pl.ds(qbase, QPW)], tv_m)
    iota = lax.iota(jnp.int32, 16)
    inf = jnp.float32(jnp.inf)
    inf16 = jnp.full((16,), inf, jnp.float32)
    ones = jnp.ones((16,), jnp.int32)
    bufs = (bufi0, bufi1, bufi2, bufi3, bufi4, bufi5, bufi6, bufi7)

    def block_body(bi, _):
        boff = pl.multiple_of(bi * 16, 16)
        qxv = qxv_m[pl.ds(boff, 16)]
        qyv = qyv_m[pl.ds(boff, 16)]
        tqv = tv_m[pl.ds(boff, 16)]
        for jp in range(16 // QG):
            qs = [(qxv[QG * jp + t], qyv[QG * jp + t]) for t in range(QG)]
            Ts = [tqv[QG * jp + t] for t in range(QG)]

            def p2(ci, offs):
                base = pl.multiple_of(ci * 16, 16)
                sxc = sxv[pl.ds(base, 16)]
                syc = syv[pl.ds(base, 16)]
                cidx = ci * 16 + iota
                out = []
                for t in range(QG):
                    dx = qs[t][0] - sxc
                    dy = qs[t][1] - syc
                    d2 = dx * dx + dy * dy
                    m = d2 <= Ts[t]
                    incl = plsc.cumsum(ones, mask=m)
                    tgt = offs[t] + incl
                    plsc.store_scatter(bufs[t], [tgt], cidx, mask=m)
                    out.append(offs[t] + plsc.all_reduce_population_count(m))
                return tuple(out)

            minus1 = jnp.full((16,), -1, jnp.int32)
            offs = lax.fori_loop(0, NCHUNK, p2, (minus1,) * QG, unroll=2)

            for t in range(QG):
                hits = offs[t][0] + 1
                bufs[t][pl.ds(hits, 16)] = jnp.full((16,), NPAD - 1, jnp.int32)
                nb = (hits + 15) // 16

                def p3(ci, carry):
                    bk, bv = carry
                    bidx = bufs[t][pl.ds(ci * 16, 16)]
                    sxg = plsc.load_gather(sxv, [bidx])
                    syg = plsc.load_gather(syv, [bidx])
                    dx = qs[t][0] - sxg
                    dy = qs[t][1] - syg
                    d2 = dx * dx + dy * dy
                    nk, nv = plsc.sort_key_val(d2, bidx, descending=True)
                    takeold = bk <= nk
                    lk = jnp.where(takeold, bk, nk)
                    lv = jnp.where(takeold, bv, nv)
                    return tuple(plsc.sort_key_val(lk, lv))

                bk0 = inf16
                bv0 = jnp.zeros((16,), jnp.int32)
                bk, bv = lax.fori_loop(0, nb, p3, (bk0, bv0))
                ob = pl.multiple_of(boff * 16 + (QG * jp + t) * 16, 16)
                idxbuf[pl.ds(ob, 16)] = bv
                d2buf[pl.ds(ob, 16)] = bk
        return 0

    lax.fori_loop(0, BPW, block_body, 0)
    pltpu.sync_copy(idxbuf, idx_hbm.at[pl.ds(qbase * 16, QPW * 16)])
    pltpu.sync_copy(d2buf, dist_hbm.at[pl.ds(qbase * 16, QPW * 16)])


@jax.jit
def _knn_sc(sxc, syc, sxq, syq, tq):
    mesh = plsc.VectorSubcoreMesh(core_axis_name="c", subcore_axis_name="s")
    f = pl.kernel(
        _knn_sc_body,
        out_type=(
            jax.ShapeDtypeStruct((NPAD * 16,), jnp.int32),
            jax.ShapeDtypeStruct((NPAD * 16,), jnp.float32),
        ),
        mesh=mesh,
        compiler_params=pltpu.CompilerParams(needs_layout_passes=False),
        scratch_types=[
            pltpu.VMEM((NPAD,), jnp.float32),
            pltpu.VMEM((NPAD,), jnp.float32),
            pltpu.VMEM((QPW,), jnp.float32),
            pltpu.VMEM((QPW,), jnp.float32),
            pltpu.VMEM((QPW,), jnp.float32),
            pltpu.VMEM((NPAD + 16,), jnp.int32),
            pltpu.VMEM((NPAD + 16,), jnp.int32),
            pltpu.VMEM((NPAD + 16,), jnp.int32),
            pltpu.VMEM((NPAD + 16,), jnp.int32),
            pltpu.VMEM((NPAD + 16,), jnp.int32),
            pltpu.VMEM((NPAD + 16,), jnp.int32),
            pltpu.VMEM((NPAD + 16,), jnp.int32),
            pltpu.VMEM((NPAD + 16,), jnp.int32),
            pltpu.VMEM((QPW * 16,), jnp.int32),
            pltpu.VMEM((QPW * 16,), jnp.float32),
        ],
    )
    return f(sxc, syc, sxq, syq, tq)


def _agg_sc_body(h_hbm, idx_hbm, d2_hbm, agg_hbm, idxv, d2v, rows0, rows1,
                 rows2, rows3, aggbuf, sem0, sem1, sem2, sem3):
    c = lax.axis_index("c")
    s = lax.axis_index("s")
    wid = s * 2 + c
    qbase = wid * QPW
    pltpu.sync_copy(idx_hbm.at[pl.ds(qbase * 16, QPW * 16)], idxv)
    pltpu.sync_copy(d2_hbm.at[pl.ds(qbase * 16, QPW * 16)], d2v)

    def gather(ql, buf, sem):
        off = jnp.minimum(ql, QPW - 1) * 16
        return pltpu.async_copy(h_hbm.at[idxv.at[pl.ds(off, 16)]], buf, sem)

    def compute(ql, buf):
        d2q = d2v[pl.ds(ql * 16, 16)]
        w = jnp.exp(jnp.float32(-10.0) * d2q)
        m0 = m1 = x0 = x1 = None
        for kk in range(16):
            wk = w[jnp.full((16,), kk, jnp.int32)]
            r0 = buf[kk, pl.ds(0, 16)] * wk
            r1 = buf[kk, pl.ds(16, 16)] * wk
            if kk == 0:
                m0, m1, x0, x1 = r0, r1, r0, r1
            else:
                m0 = m0 + r0
                m1 = m1 + r1
                x0 = jnp.maximum(x0, r0)
                x1 = jnp.maximum(x1, r1)
        scale = jnp.float32(1.0 / 16.0)
        ob = ql * 64
        aggbuf[pl.ds(ob, 16)] = m0 * scale
        aggbuf[pl.ds(ob + 16, 16)] = m1 * scale
        aggbuf[pl.ds(ob + 32, 16)] = x0
        aggbuf[pl.ds(ob + 48, 16)] = x1

    allrows = (rows0, rows1, rows2, rows3)
    allsems = (sem0, sem1, sem2, sem3)
    for b in range(4):
        gather(b, allrows[b], allsems[b])

    def qbody(i, _):
        q0 = i * 4
        for b in range(4):
            pltpu.make_async_copy(h_hbm.at[idxv.at[pl.ds(0, 16)]],
                                  allrows[b], allsems[b]).wait()
            compute(q0 + b, allrows[b])
            gather(q0 + b + 4, allrows[b], allsems[b])
        return 0

    lax.fori_loop(0, QPW // 4, qbody, 0)
    for b in range(4):
        pltpu.make_async_copy(h_hbm.at[idxv.at[pl.ds(0, 16)]],
                              allrows[b], allsems[b]).wait()
    pltpu.sync_copy(aggbuf, agg_hbm.at[pl.ds(qbase * 64, QPW * 64)])


@jax.jit
def _agg_sc(hp, idx_flat, d2_flat):
    mesh = plsc.VectorSubcoreMesh(core_axis_name="c", subcore_axis_name="s")
    f = pl.kernel(
        _agg_sc_body,
        out_type=jax.ShapeDtypeStruct((NPAD * 2 * HID,), jnp.float32),
        mesh=mesh,
        compiler_params=pltpu.CompilerParams(needs_layout_passes=False),
        scratch_types=[
            pltpu.VMEM((QPW * 16,), jnp.int32),
            pltpu.VMEM((QPW * 16,), jnp.float32),
            pltpu.VMEM((16, 128), jnp.float32),
            pltpu.VMEM((16, 128), jnp.float32),
            pltpu.VMEM((16, 128), jnp.float32),
            pltpu.VMEM((16, 128), jnp.float32),
            pltpu.VMEM((QPW * 2 * HID,), jnp.float32),
            pltpu.SemaphoreType.DMA,
            pltpu.SemaphoreType.DMA,
            pltpu.SemaphoreType.DMA,
            pltpu.SemaphoreType.DMA,
        ],
    )
    return f(hp, idx_flat, d2_flat)


def kernel(x, Ws, bs, Wh, bh, Wo1, Wo2, bo2, W20, b20, W21, b21, W22, b22,
           W23, b23, W30, b30, W31, b31, W32, b32, W33, b33):
    s, h = _space_hidden(x, Ws, bs, Wh, bh)
    pad = jnp.full((NPAD - N,), PAD_COORD, jnp.float32)
    zpad = jnp.zeros((NPAD - N,), jnp.float32)
    sxc = jnp.concatenate([s[:, 0], pad])
    syc = jnp.concatenate([s[:, 1], pad])
    sxq = jnp.concatenate([s[:, 0], zpad])
    syq = jnp.concatenate([s[:, 1], zpad])
    tq = _thresh_tc(sxq, syq, sxc, syc).reshape(NPAD)
    idx_flat, d2_flat = _knn_sc(sxc, syc, sxq, syq, tq)
    idx = idx_flat.reshape(NPAD, K)[:N]
    hp = jnp.pad(h, ((0, NPAD - N), (0, 128 - HID)))
    aggp = _agg_sc(hp, idx_flat, d2_flat).reshape(NPAD, 2 * HID)
    xp = jnp.pad(x, ((0, NP - N), (0, 0)))
    ids_p, p4_p = _mlp(xp, aggp, Wo1, Wo2, bo2, W20, b20, W21, b21, W22, b22,
                       W23, b23, W30, b30, W31, b31, W32, b32, W33, b33)
    cand_ids = ids_p[:N]
    cand_p4 = p4_p[:N]
    src = idx.reshape(-1)
    dst = jnp.repeat(jnp.arange(idx.shape[0]), idx.shape[1])
    edge_index = jnp.stack([src, dst])
    return (cand_ids, cand_p4, edge_index)
